# baseline (speedup 1.0000x reference)
import functools

import jax
import jax.numpy as jnp
from jax import lax
from jax.experimental import pallas as pl
from jax.experimental.pallas import tpu as pltpu
from jax.experimental.pallas import tpu_sc as plsc

NG = 64
DELTA = 20.0 / (NG - 1)
COEFF = -0.5 / (DELTA * DELTA)
LANES = 16
NW = 32
SUB = 100
CHUNK = 400


@functools.lru_cache(maxsize=None)
def _build(E):
    per_w = E // NW
    n_chunks = per_w // CHUNK
    n_sub = CHUNK // SUB
    mesh = plsc.VectorSubcoreMesh(
        core_axis_name="c", subcore_axis_name="s", num_cores=2, num_subcores=16
    )

    @functools.partial(
        pl.kernel,
        out_type=jax.ShapeDtypeStruct((E, 2 * NG), jnp.float32),
        mesh=mesh,
        compiler_params=pltpu.CompilerParams(
            use_tc_tiling_on_sc=False, needs_layout_passes=False
        ),
        scratch_types=[
            pltpu.VMEM((CHUNK,), jnp.float32),
            pltpu.VMEM((n_sub, SUB), jnp.int32),
            pltpu.VMEM((CHUNK, NG), jnp.float32),
            pltpu.VMEM((CHUNK, NG), jnp.float32),
            pltpu.SemaphoreType.DMA,
        ],
    )
    def sc_kernel(len_hbm, idx_hbm, table_hbm, out_hbm,
                  len_v, idx_v, emb_v, rbf_v, sem):
        wid = lax.axis_index("s") * 2 + lax.axis_index("c")
        lane = lax.iota(jnp.int32, LANES)
        lane_f = lane.astype(jnp.float32)
        offs = [lane_f * DELTA + (k * LANES * DELTA) for k in range(4)]

        def chunk_body(c, carry):
            base = wid * per_w + c * CHUNK
            pltpu.sync_copy(len_hbm.at[pl.ds(base, CHUNK)], len_v)
            pltpu.sync_copy(idx_hbm.at[pl.ds(base // SUB, n_sub)], idx_v)
            gathers = [
                pltpu.async_copy(
                    table_hbm.at[idx_v.at[j]],
                    emb_v.at[pl.ds(j * SUB, SUB)],
                    sem,
                )
                for j in range(n_sub)
            ]

            def e_body(e, carry2):
                d16 = len_v[pl.ds(e * LANES, LANES)]
                for i in range(LANES):
                    d = lax.gather(
                        d16, jnp.full((LANES, 1), i, jnp.int32),
                        dimension_numbers=lax.GatherDimensionNumbers(
                            offset_dims=(), collapsed_slice_dims=(0,),
                            start_index_map=(0,)),
                        slice_sizes=(1,),
                        mode=lax.GatherScatterMode.PROMISE_IN_BOUNDS)
                    for k in range(4):
                        t = d - offs[k]
                        rbf_v[e * LANES + i, pl.ds(k * LANES, LANES)] = (
                            jnp.exp(COEFF * (t * t))
                        )
                return carry2

            lax.fori_loop(0, CHUNK // LANES, e_body, 0, unroll=False)
            for g_ in gathers:
                g_.wait()
            pltpu.sync_copy(rbf_v, out_hbm.at[pl.ds(base, CHUNK), pl.ds(0, NG)])
            pltpu.sync_copy(emb_v, out_hbm.at[pl.ds(base, CHUNK), pl.ds(NG, NG)])
            return carry

        lax.fori_loop(0, n_chunks, chunk_body, 0, unroll=False)

    return sc_kernel


def kernel(edge_length, edge_type, bond_emb_weight):
    E = edge_length.shape[0]
    lengths = edge_length.reshape(E)
    idx = edge_type.astype(jnp.int32).reshape(E // SUB, SUB)
    fn = _build(E)
    return fn(lengths, idx, bond_emb_weight)


# speedup vs baseline: 1.2198x; 1.2198x over previous
import functools

import jax
import jax.numpy as jnp
from jax import lax
from jax.experimental import pallas as pl
from jax.experimental.pallas import tpu as pltpu
from jax.experimental.pallas import tpu_sc as plsc

NG = 64
DELTA = 20.0 / (NG - 1)
COEFF = -0.5 / (DELTA * DELTA)
LANES = 16
NW = 32
SUB = 100
CHUNK = 400


@functools.lru_cache(maxsize=None)
def _build(E):
    per_w = E // NW
    n_chunks = per_w // CHUNK
    n_sub = CHUNK // SUB
    mesh = plsc.VectorSubcoreMesh(
        core_axis_name="c", subcore_axis_name="s", num_cores=2, num_subcores=16
    )

    @functools.partial(
        pl.kernel,
        out_type=jax.ShapeDtypeStruct((E, 2 * NG), jnp.float32),
        mesh=mesh,
        compiler_params=pltpu.CompilerParams(
            use_tc_tiling_on_sc=False, needs_layout_passes=False
        ),
        scratch_types=[
            pltpu.VMEM((CHUNK,), jnp.float32),
            pltpu.VMEM((n_sub, SUB), jnp.int32),
            pltpu.VMEM((CHUNK, NG), jnp.float32),
            pltpu.VMEM((CHUNK, NG), jnp.float32),
            pltpu.SemaphoreType.DMA,
        ],
    )
    def sc_kernel(len_hbm, idx_hbm, table_hbm, out_hbm,
                  len_v, idx_v, emb_v, rbf_v, sem):
        wid = lax.axis_index("s") * 2 + lax.axis_index("c")
        lane = lax.iota(jnp.int32, LANES)
        lane_f = lane.astype(jnp.float32)
        offs = [lane_f * DELTA + (k * LANES * DELTA) for k in range(4)]

        def chunk_body(c, carry):
            base = wid * per_w + c * CHUNK
            pltpu.sync_copy(len_hbm.at[pl.ds(base, CHUNK)], len_v)
            pltpu.sync_copy(idx_hbm.at[pl.ds(base // SUB, n_sub)], idx_v)
            gathers = [
                pltpu.async_copy(
                    table_hbm.at[idx_v.at[j]],
                    emb_v.at[pl.ds(j * SUB, SUB)],
                    sem,
                )
                for j in range(n_sub)
            ]

            def e_body(e, carry2):
                d16 = len_v[pl.ds(e * LANES, LANES)]
                row = lane + e * LANES
                for g in range(NG):
                    t = d16 - (g * DELTA)
                    v = jnp.exp(COEFF * (t * t))
                    col = jnp.full((LANES,), g, jnp.int32)
                    plsc.store_scatter(rbf_v, [row, col], v)
                return carry2

            lax.fori_loop(0, CHUNK // LANES, e_body, 0, unroll=False)
            for g_ in gathers:
                g_.wait()
            pltpu.sync_copy(rbf_v, out_hbm.at[pl.ds(base, CHUNK), pl.ds(0, NG)])
            pltpu.sync_copy(emb_v, out_hbm.at[pl.ds(base, CHUNK), pl.ds(NG, NG)])
            return carry

        lax.fori_loop(0, n_chunks, chunk_body, 0, unroll=False)

    return sc_kernel


def kernel(edge_length, edge_type, bond_emb_weight):
    E = edge_length.shape[0]
    lengths = edge_length.reshape(E)
    idx = edge_type.astype(jnp.int32).reshape(E // SUB, SUB)
    fn = _build(E)
    return fn(lengths, idx, bond_emb_weight)


# speedup vs baseline: 1.3255x; 1.0867x over previous
import functools

import jax
import jax.numpy as jnp
from jax import lax
from jax.experimental import pallas as pl
from jax.experimental.pallas import tpu as pltpu
from jax.experimental.pallas import tpu_sc as plsc

NG = 64
DELTA = 20.0 / (NG - 1)
COEFF = -0.5 / (DELTA * DELTA)
LANES = 16
NW = 32
SUB = 80
CHUNK = 400


@functools.lru_cache(maxsize=None)
def _build(E):
    per_w = E // NW
    n_chunks = per_w // CHUNK
    n_sub = CHUNK // SUB
    n_rows = per_w // SUB
    mesh = plsc.VectorSubcoreMesh(
        core_axis_name="c", subcore_axis_name="s", num_cores=2, num_subcores=16
    )

    @functools.partial(
        pl.kernel,
        out_type=jax.ShapeDtypeStruct((E, 2 * NG), jnp.float32),
        mesh=mesh,
        compiler_params=pltpu.CompilerParams(
            use_tc_tiling_on_sc=False, needs_layout_passes=False
        ),
        scratch_types=[
            pltpu.VMEM((per_w,), jnp.float32),
            pltpu.VMEM((n_rows, SUB), jnp.int32),
            pltpu.VMEM((CHUNK, NG), jnp.float32),
            pltpu.VMEM((CHUNK, NG), jnp.float32),
            pltpu.SemaphoreType.DMA,
        ],
    )
    def sc_kernel(len_hbm, idx_hbm, table_hbm, out_hbm,
                  len_v, idx_v, emb_v, rbf_v, sem):
        wid = lax.axis_index("s") * 2 + lax.axis_index("c")
        lane = lax.iota(jnp.int32, LANES)

        pltpu.sync_copy(len_hbm.at[pl.ds(wid * per_w, per_w)], len_v)
        pltpu.sync_copy(idx_hbm.at[pl.ds(wid * n_rows, n_rows)], idx_v)

        def chunk_body(c, carry):
            base = wid * per_w + c * CHUNK
            gathers = [
                pltpu.async_copy(
                    table_hbm.at[idx_v.at[c * n_sub + j]],
                    emb_v.at[pl.ds(j * SUB, SUB)],
                    sem,
                )
                for j in range(n_sub)
            ]

            def e_body(e, carry2):
                d16 = len_v[pl.ds(c * CHUNK + e * LANES, LANES)]
                row = lane + e * LANES
                for g in range(NG):
                    t = d16 - (g * DELTA)
                    v = jnp.exp(COEFF * (t * t))
                    col = jnp.full((LANES,), g, jnp.int32)
                    plsc.store_scatter(rbf_v, [row, col], v)
                return carry2

            lax.fori_loop(0, CHUNK // LANES, e_body, 0, unroll=False)
            for g_ in gathers:
                g_.wait()
            pltpu.sync_copy(rbf_v, out_hbm.at[pl.ds(base, CHUNK), pl.ds(0, NG)])
            pltpu.sync_copy(emb_v, out_hbm.at[pl.ds(base, CHUNK), pl.ds(NG, NG)])
            return carry

        lax.fori_loop(0, n_chunks, chunk_body, 0, unroll=False)

    return sc_kernel


def kernel(edge_length, edge_type, bond_emb_weight):
    E = edge_length.shape[0]
    lengths = edge_length.reshape(E)
    idx = edge_type.astype(jnp.int32).reshape(E // SUB, SUB)
    fn = _build(E)
    return fn(lengths, idx, bond_emb_weight)


# speedup vs baseline: 1.4863x; 1.1213x over previous
import functools

import jax
import jax.numpy as jnp
from jax import lax
from jax.experimental import pallas as pl
from jax.experimental.pallas import tpu as pltpu
from jax.experimental.pallas import tpu_sc as plsc

NG = 64
DELTA = 20.0 / (NG - 1)
COEFF = -0.5 / (DELTA * DELTA)
LANES = 16
NW = 32
SUB = 80
CHUNK = 400


@functools.lru_cache(maxsize=None)
def _build(E):
    per_w = E // NW
    n_chunks = per_w // CHUNK
    n_sub = CHUNK // SUB
    n_rows = per_w // SUB
    mesh = plsc.VectorSubcoreMesh(
        core_axis_name="c", subcore_axis_name="s", num_cores=2, num_subcores=16
    )

    @functools.partial(
        pl.kernel,
        out_type=jax.ShapeDtypeStruct((E, 2 * NG), jnp.float32),
        mesh=mesh,
        compiler_params=pltpu.CompilerParams(
            use_tc_tiling_on_sc=False, needs_layout_passes=False
        ),
        scratch_types=[
            pltpu.VMEM((per_w,), jnp.float32),
            pltpu.VMEM((n_rows, SUB), jnp.int32),
            pltpu.VMEM((CHUNK, NG), jnp.float32),
            pltpu.VMEM((CHUNK, NG), jnp.float32),
            pltpu.SemaphoreType.DMA,
        ],
    )
    def sc_kernel(len_hbm, idx_hbm, table_hbm, out_hbm,
                  len_v, idx_v, emb_v, rbf_v, sem):
        wid = lax.axis_index("s") * 2 + lax.axis_index("c")
        lane = lax.iota(jnp.int32, LANES)
        lane_f = lane.astype(jnp.float32)
        offs = [lane_f * DELTA + (k * LANES * DELTA) for k in range(4)]

        pltpu.sync_copy(len_hbm.at[pl.ds(wid * per_w, per_w)], len_v)
        pltpu.sync_copy(idx_hbm.at[pl.ds(wid * n_rows, n_rows)], idx_v)

        def chunk_body(c, carry):
            base = wid * per_w + c * CHUNK
            gathers = [
                pltpu.async_copy(
                    table_hbm.at[idx_v.at[c * n_sub + j]],
                    emb_v.at[pl.ds(j * SUB, SUB)],
                    sem,
                )
                for j in range(n_sub)
            ]

            def e_body(e, carry2):
                d16 = len_v[pl.ds(c * CHUNK + e * LANES, LANES)]
                for i in range(LANES):
                    d = lax.gather(
                        d16, jnp.full((LANES, 1), i, jnp.int32),
                        dimension_numbers=lax.GatherDimensionNumbers(
                            offset_dims=(), collapsed_slice_dims=(0,),
                            start_index_map=(0,)),
                        slice_sizes=(1,),
                        mode=lax.GatherScatterMode.PROMISE_IN_BOUNDS)
                    for k in range(4):
                        t = d - offs[k]
                        rbf_v[e * LANES + i, pl.ds(k * LANES, LANES)] = (
                            jnp.exp(COEFF * (t * t))
                        )
                return carry2

            lax.fori_loop(0, CHUNK // LANES, e_body, 0, unroll=False)
            for g_ in gathers:
                g_.wait()
            pltpu.sync_copy(rbf_v, out_hbm.at[pl.ds(base, CHUNK), pl.ds(0, NG)])
            pltpu.sync_copy(emb_v, out_hbm.at[pl.ds(base, CHUNK), pl.ds(NG, NG)])
            return carry

        lax.fori_loop(0, n_chunks, chunk_body, 0, unroll=False)

    return sc_kernel


def kernel(edge_length, edge_type, bond_emb_weight):
    E = edge_length.shape[0]
    lengths = edge_length.reshape(E)
    idx = edge_type.astype(jnp.int32).reshape(E // SUB, SUB)
    fn = _build(E)
    return fn(lengths, idx, bond_emb_weight)
